# baseline (speedup 1.0000x reference)
import functools

import jax
import jax.numpy as jnp
from jax import lax
from jax.experimental import pallas as pl
from jax.experimental.pallas import tpu as pltpu
from jax.experimental.pallas import tpu_sc as plsc

N = 1_000_000
D = 32
Q = 8
NC = 2
NS = 16
NW = NC * NS
L = 16

CHUNK = 1024
NCHUNKS = (N + CHUNK - 1) // CHUNK
TPC = 31
TILE = 64
EPS = 1e-8
EPS2 = EPS * EPS
NEG_INF = float("-inf")


def _sc_topk_kernel(keys_flat, qsplat, out_t, out_i, buf0, buf1, qv, res_t,
                    res_i, sem0, sem1):
    cid = lax.axis_index("c")
    sid = lax.axis_index("s")
    wid = cid * NS + sid

    pltpu.sync_copy(qsplat, qv)

    def chunk_off(t):
        c = jnp.minimum(wid + NW * t, NCHUNKS - 1)
        return jnp.minimum(c * CHUNK, N - CHUNK) * D

    bufs = (buf0, buf1)
    sems = (sem0, sem1)

    pltpu.async_copy(keys_flat.at[pl.ds(chunk_off(0), CHUNK * D)], buf0, sem0)

    iota = lax.iota(jnp.int32, L)
    lane_off = iota * D

    def process_chunk(t, buf, carry):
        best_t, best_i = carry
        row0 = chunk_off(t) // D

        def tile_body(tile, carry):
            best_t, best_i = carry
            base = tile * (TILE * D)
            pre = [base + j * (L * D) + lane_off for j in range(TILE // L)]

            accs0 = tuple(jnp.zeros((L,), jnp.float32)
                          for _ in range(len(pre) * (Q + 1)))

            accs = list(accs0)
            for d in range(D):
                v = [plsc.load_gather(buf, [p + d]) for p in pre]
                for q in range(Q):
                    s = qv[pl.ds((q * D + d) * L, L)]
                    for j in range(len(pre)):
                        accs[j * (Q + 1) + q] = accs[j * (Q + 1) + q] + v[j] * s
                for j in range(len(pre)):
                    accs[j * (Q + 1) + Q] = accs[j * (Q + 1) + Q] + v[j] * v[j]

            best_t = list(best_t)
            best_i = list(best_i)
            for j in range(len(pre)):
                nrm = accs[j * (Q + 1) + Q]
                rcp = 1.0 / jnp.maximum(nrm, EPS2)
                idx_vec = (row0 + tile * TILE + j * L) + iota
                for q in range(Q):
                    dot = accs[j * (Q + 1) + q]
                    tval = dot * jnp.abs(dot) * rcp
                    better = tval > best_t[q]
                    best_t[q] = jnp.where(better, tval, best_t[q])
                    best_i[q] = jnp.where(better, idx_vec, best_i[q])
            return (tuple(best_t), tuple(best_i))

        return lax.fori_loop(0, CHUNK // TILE, tile_body, (best_t, best_i))

    best_t = tuple(jnp.full((L,), NEG_INF, jnp.float32) for _ in range(Q))
    best_i = tuple(jnp.zeros((L,), jnp.int32) for _ in range(Q))
    carry = (best_t, best_i)

    def outer(i, carry):
        for b in range(2):
            t = 2 * i + b
            nxt = jnp.minimum(t + 1, TPC)
            pltpu.async_copy(
                keys_flat.at[pl.ds(chunk_off(nxt), CHUNK * D)],
                bufs[1 - b], sems[1 - b])
            pltpu.make_async_copy(
                keys_flat.at[pl.ds(chunk_off(t), CHUNK * D)],
                bufs[b], sems[b]).wait()
            carry = process_chunk(jnp.minimum(t, TPC - 1), bufs[b], carry)
        return carry

    carry = lax.fori_loop(0, 16, outer, carry)
    pltpu.make_async_copy(
        keys_flat.at[pl.ds(chunk_off(TPC), CHUNK * D)], bufs[0], sems[0]
    ).wait()

    best_t, best_i = carry
    for q in range(Q):
        res_t[q, :] = best_t[q]
        res_i[q, :] = best_i[q]
    pltpu.sync_copy(res_t, out_t.at[wid])
    pltpu.sync_copy(res_i, out_i.at[wid])


def _run_sc(keys_flat, qsplat):
    mesh = plsc.VectorSubcoreMesh(core_axis_name="c", subcore_axis_name="s",
                                  num_cores=NC, num_subcores=NS)
    f = pl.kernel(
        _sc_topk_kernel,
        out_type=(
            jax.ShapeDtypeStruct((NW, Q, L), jnp.float32),
            jax.ShapeDtypeStruct((NW, Q, L), jnp.int32),
        ),
        mesh=mesh,
        scratch_types=[
            pltpu.VMEM((CHUNK * D,), jnp.float32),
            pltpu.VMEM((CHUNK * D,), jnp.float32),
            pltpu.VMEM((Q * D * L,), jnp.float32),
            pltpu.VMEM((Q, L), jnp.float32),
            pltpu.VMEM((Q, L), jnp.int32),
            pltpu.SemaphoreType.DMA,
            pltpu.SemaphoreType.DMA,
        ],
        compiler_params=pltpu.CompilerParams(
            needs_layout_passes=False, use_tc_tiling_on_sc=False),
    )
    return f(keys_flat, qsplat)


@jax.jit
def kernel(queries, keys):
    qn = queries / jnp.maximum(
        jnp.linalg.norm(queries, axis=-1, keepdims=True), EPS)
    qsplat = jnp.broadcast_to(qn.reshape(Q, D, 1), (Q, D, L)).reshape(-1)
    t_c, i_c = _run_sc(keys.reshape(-1), qsplat)
    sims = jnp.sign(t_c) * jnp.sqrt(jnp.abs(t_c))
    sims = sims.transpose(1, 0, 2).reshape(Q, NW * L)
    idx = i_c.transpose(1, 0, 2).reshape(Q, NW * L)
    vals = jnp.max(sims, axis=1)
    at_max = sims == vals[:, None]
    best_idx = jnp.min(jnp.where(at_max, idx, N), axis=1)
    return vals, best_idx.astype(jnp.int32)


# speedup vs baseline: 1.2198x; 1.2198x over previous
import functools

import jax
import jax.numpy as jnp
from jax import lax
from jax.experimental import pallas as pl
from jax.experimental.pallas import tpu as pltpu
from jax.experimental.pallas import tpu_sc as plsc

N = 1_000_000
D = 32
Q = 8
NC = 2
NS = 16
NW = NC * NS
L = 16

CHUNK = 1024
TPC = 3
N_SC = NW * TPC * CHUNK
N_TC = N - N_SC
BK = 8192
G_TC = (N_TC + BK - 1) // BK
TILE = 64
EPS = 1e-8
EPS2 = EPS * EPS
NEG_INF = float("-inf")


def _tc_body(qn_ref, keys_ref, out_v, out_i, best_v, best_i):
    pi = pl.program_id(0)

    @pl.when(pi == 0)
    def _init():
        best_v[...] = jnp.full((Q, 1), NEG_INF, jnp.float32)
        best_i[...] = jnp.zeros((Q, 1), jnp.int32)

    k = keys_ref[...]
    qn = qn_ref[...]
    d = lax.dot_general(qn, k, (((1,), (1,)), ((), ())),
                        preferred_element_type=jnp.float32)
    ksq = k * k
    ones = jnp.ones((Q, D), jnp.float32)
    s8 = lax.dot_general(ones, ksq, (((1,), (1,)), ((), ())),
                         precision=lax.Precision.HIGHEST,
                         preferred_element_type=jnp.float32)
    rs = lax.rsqrt(jnp.maximum(s8[0:1], EPS2))
    sims = d * rs

    row = pi * BK + lax.broadcasted_iota(jnp.int32, (Q, BK), 1)
    sims = jnp.where(row < N_TC, sims, NEG_INF)

    m = jnp.max(sims, axis=1, keepdims=True)
    cand = jnp.where(sims == m, row, N)
    ci = jnp.min(cand, axis=1, keepdims=True)

    upd = m > best_v[...]
    best_v[...] = jnp.where(upd, m, best_v[...])
    best_i[...] = jnp.where(upd, ci, best_i[...])

    @pl.when(pi == G_TC - 1)
    def _out():
        out_v[...] = best_v[...]
        out_i[...] = best_i[...]


def _run_tc(qn, keys):
    return pl.pallas_call(
        _tc_body,
        grid=(G_TC,),
        in_specs=[
            pl.BlockSpec((Q, D), lambda i: (0, 0)),
            pl.BlockSpec((BK, D), lambda i: (i, 0)),
        ],
        out_specs=[
            pl.BlockSpec((Q, 1), lambda i: (0, 0)),
            pl.BlockSpec((Q, 1), lambda i: (0, 0)),
        ],
        out_shape=[
            jax.ShapeDtypeStruct((Q, 1), jnp.float32),
            jax.ShapeDtypeStruct((Q, 1), jnp.int32),
        ],
        scratch_shapes=[
            pltpu.VMEM((Q, 1), jnp.float32),
            pltpu.VMEM((Q, 1), jnp.int32),
        ],
    )(qn, keys)


def _sc_body(keys, qsplat, out_t, out_i, buf0, buf1, qv, res_t, res_i,
             sem0, sem1):
    cid = lax.axis_index("c")
    sid = lax.axis_index("s")
    wid = cid * NS + sid

    pltpu.sync_copy(qsplat, qv)

    def row0(t):
        return N_TC + (wid + NW * t) * CHUNK

    iota = lax.iota(jnp.int32, L)

    def process_chunk(t, buf, carry):
        base_row = row0(t)

        def tile_body(tile, carry):
            best_t, best_i = carry
            rows = [tile * TILE + j * L + iota for j in range(TILE // L)]
            nj = len(rows)

            accs = [jnp.zeros((L,), jnp.float32) for _ in range(nj * (Q + 1))]
            for d in range(D):
                col = jnp.full((L,), d, jnp.int32)
                v = [plsc.load_gather(buf, [r, col]) for r in rows]
                for q in range(Q):
                    s = qv[pl.ds((q * D + d) * L, L)]
                    for j in range(nj):
                        accs[j * (Q + 1) + q] = accs[j * (Q + 1) + q] + v[j] * s
                for j in range(nj):
                    accs[j * (Q + 1) + Q] = accs[j * (Q + 1) + Q] + v[j] * v[j]

            best_t = list(best_t)
            best_i = list(best_i)
            for j in range(nj):
                rcp = 1.0 / jnp.maximum(accs[j * (Q + 1) + Q], EPS2)
                idx_vec = base_row + rows[j]
                for q in range(Q):
                    dot = accs[j * (Q + 1) + q]
                    tval = dot * jnp.abs(dot) * rcp
                    better = tval > best_t[q]
                    best_t[q] = jnp.where(better, tval, best_t[q])
                    best_i[q] = jnp.where(better, idx_vec, best_i[q])
            return (tuple(best_t), tuple(best_i))

        return lax.fori_loop(0, CHUNK // TILE, tile_body, carry)

    best_t = tuple(jnp.full((L,), NEG_INF, jnp.float32) for _ in range(Q))
    best_i = tuple(jnp.zeros((L,), jnp.int32) for _ in range(Q))
    carry = (best_t, best_i)

    pltpu.async_copy(keys.at[pl.ds(row0(0), CHUNK)], buf0, sem0)
    pltpu.async_copy(keys.at[pl.ds(row0(1), CHUNK)], buf1, sem1)
    pltpu.make_async_copy(keys.at[pl.ds(row0(0), CHUNK)], buf0, sem0).wait()
    carry = process_chunk(0, buf0, carry)
    pltpu.async_copy(keys.at[pl.ds(row0(2), CHUNK)], buf0, sem0)
    pltpu.make_async_copy(keys.at[pl.ds(row0(1), CHUNK)], buf1, sem1).wait()
    carry = process_chunk(1, buf1, carry)
    pltpu.make_async_copy(keys.at[pl.ds(row0(2), CHUNK)], buf0, sem0).wait()
    carry = process_chunk(2, buf0, carry)

    best_t, best_i = carry
    for q in range(Q):
        res_t[q, :] = best_t[q]
        res_i[q, :] = best_i[q]
    pltpu.sync_copy(res_t, out_t.at[wid])
    pltpu.sync_copy(res_i, out_i.at[wid])


def _run_sc(keys, qsplat):
    mesh = plsc.VectorSubcoreMesh(core_axis_name="c", subcore_axis_name="s",
                                  num_cores=NC, num_subcores=NS)
    f = pl.kernel(
        _sc_body,
        out_type=(
            jax.ShapeDtypeStruct((NW, Q, L), jnp.float32),
            jax.ShapeDtypeStruct((NW, Q, L), jnp.int32),
        ),
        mesh=mesh,
        scratch_types=[
            pltpu.VMEM((CHUNK, D), jnp.float32),
            pltpu.VMEM((CHUNK, D), jnp.float32),
            pltpu.VMEM((Q * D * L,), jnp.float32),
            pltpu.VMEM((Q, L), jnp.float32),
            pltpu.VMEM((Q, L), jnp.int32),
            pltpu.SemaphoreType.DMA,
            pltpu.SemaphoreType.DMA,
        ],
        compiler_params=pltpu.CompilerParams(
            needs_layout_passes=False, use_tc_tiling_on_sc=False),
    )
    return f(keys, qsplat)


@jax.jit
def kernel(queries, keys):
    qn = queries / jnp.maximum(
        jnp.linalg.norm(queries, axis=-1, keepdims=True), EPS)
    qsplat = jnp.broadcast_to(qn.reshape(Q, D, 1), (Q, D, L)).reshape(-1)

    t_c, i_c = _run_sc(keys, qsplat)
    tc_v, tc_i = _run_tc(qn, keys)

    sc_sims = jnp.sign(t_c) * jnp.sqrt(jnp.abs(t_c))
    sc_sims = sc_sims.transpose(1, 0, 2).reshape(Q, NW * L)
    sc_idx = i_c.transpose(1, 0, 2).reshape(Q, NW * L)
    sims = jnp.concatenate([sc_sims, tc_v], axis=1)
    idx = jnp.concatenate([sc_idx, tc_i], axis=1)
    vals = jnp.max(sims, axis=1)
    at_max = sims == vals[:, None]
    best_idx = jnp.min(jnp.where(at_max, idx, N), axis=1)
    return vals, best_idx.astype(jnp.int32)


# speedup vs baseline: 1.4037x; 1.1507x over previous
import functools

import jax
import jax.numpy as jnp
from jax import lax
from jax.experimental import pallas as pl
from jax.experimental.pallas import tpu as pltpu
from jax.experimental.pallas import tpu_sc as plsc

N = 1_000_000
D = 32
Q = 8
NC = 2
NS = 16
NW = NC * NS
L = 16

CHUNK = 1024
TPC = 3
N_SC = NW * TPC * CHUNK
N_TC = N - N_SC
TILE = 64
EPS = 1e-8
EPS2 = EPS * EPS
NEG_INF = float("-inf")


P4 = 4
W4 = P4 * D
NR = N // P4
BK4 = 2048
G_TC = (N_TC // P4 + BK4 - 1) // BK4
QP = Q * P4


def _tc_body(qn4_ref, ones4_ref, keys_ref, out_v, out_i, best_v, best_i):
    pi = pl.program_id(0)

    @pl.when(pi == 0)
    def _init():
        best_v[...] = jnp.full((QP, 1), NEG_INF, jnp.float32)
        best_i[...] = jnp.zeros((QP, 1), jnp.int32)

    kr = keys_ref[...]
    d4 = lax.dot_general(qn4_ref[...], kr, (((1,), (1,)), ((), ())),
                         preferred_element_type=jnp.float32)
    ksq = kr * kr
    s4 = lax.dot_general(ones4_ref[...], ksq, (((1,), (1,)), ((), ())),
                         preferred_element_type=jnp.float32)
    rs = lax.rsqrt(jnp.maximum(s4, EPS2))
    sims = d4 * jnp.tile(rs, (Q, 1))

    col = lax.broadcasted_iota(jnp.int32, (QP, BK4), 1)
    prow = lax.broadcasted_iota(jnp.int32, (QP, BK4), 0) % P4
    row = pi * (BK4 * P4) + col * P4 + prow
    sims = jnp.where(row < N_TC, sims, NEG_INF)

    m = jnp.max(sims, axis=1, keepdims=True)
    cand = jnp.where(sims == m, row, N)
    ci = jnp.min(cand, axis=1, keepdims=True)

    upd = m > best_v[...]
    best_v[...] = jnp.where(upd, m, best_v[...])
    best_i[...] = jnp.where(upd, ci, best_i[...])

    @pl.when(pi == G_TC - 1)
    def _out():
        out_v[...] = best_v[...]
        out_i[...] = best_i[...]


def _run_tc(qn4, ones4, keys_r):
    return pl.pallas_call(
        _tc_body,
        grid=(G_TC,),
        in_specs=[
            pl.BlockSpec((QP, W4), lambda i: (0, 0)),
            pl.BlockSpec((P4, W4), lambda i: (0, 0)),
            pl.BlockSpec((BK4, W4), lambda i: (i, 0)),
        ],
        out_specs=[
            pl.BlockSpec((QP, 1), lambda i: (0, 0)),
            pl.BlockSpec((QP, 1), lambda i: (0, 0)),
        ],
        out_shape=[
            jax.ShapeDtypeStruct((QP, 1), jnp.float32),
            jax.ShapeDtypeStruct((QP, 1), jnp.int32),
        ],
        scratch_shapes=[
            pltpu.VMEM((QP, 1), jnp.float32),
            pltpu.VMEM((QP, 1), jnp.int32),
        ],
    )(qn4, ones4, keys_r)


def _sc_body(keys, qsplat, out_t, out_i, buf0, buf1, qv, res_t, res_i,
             sem0, sem1):
    cid = lax.axis_index("c")
    sid = lax.axis_index("s")
    wid = cid * NS + sid

    pltpu.sync_copy(qsplat, qv)

    def row0(t):
        return N_TC + (wid + NW * t) * CHUNK

    iota = lax.iota(jnp.int32, L)

    def process_chunk(t, buf, carry):
        base_row = row0(t)

        def tile_body(tile, carry):
            best_t, best_i = carry
            rows = [tile * TILE + j * L + iota for j in range(TILE // L)]
            nj = len(rows)

            accs = [jnp.zeros((L,), jnp.float32) for _ in range(nj * (Q + 1))]
            for d in range(D):
                col = jnp.full((L,), d, jnp.int32)
                v = [plsc.load_gather(buf, [r, col]) for r in rows]
                for q in range(Q):
                    s = qv[pl.ds((q * D + d) * L, L)]
                    for j in range(nj):
                        accs[j * (Q + 1) + q] = accs[j * (Q + 1) + q] + v[j] * s
                for j in range(nj):
                    accs[j * (Q + 1) + Q] = accs[j * (Q + 1) + Q] + v[j] * v[j]

            best_t = list(best_t)
            best_i = list(best_i)
            for j in range(nj):
                rcp = 1.0 / jnp.maximum(accs[j * (Q + 1) + Q], EPS2)
                idx_vec = base_row + rows[j]
                for q in range(Q):
                    dot = accs[j * (Q + 1) + q]
                    tval = dot * jnp.abs(dot) * rcp
                    better = tval > best_t[q]
                    best_t[q] = jnp.where(better, tval, best_t[q])
                    best_i[q] = jnp.where(better, idx_vec, best_i[q])
            return (tuple(best_t), tuple(best_i))

        return lax.fori_loop(0, CHUNK // TILE, tile_body, carry)

    best_t = tuple(jnp.full((L,), NEG_INF, jnp.float32) for _ in range(Q))
    best_i = tuple(jnp.zeros((L,), jnp.int32) for _ in range(Q))
    carry = (best_t, best_i)

    pltpu.async_copy(keys.at[pl.ds(row0(0), CHUNK)], buf0, sem0)
    pltpu.async_copy(keys.at[pl.ds(row0(1), CHUNK)], buf1, sem1)
    pltpu.make_async_copy(keys.at[pl.ds(row0(0), CHUNK)], buf0, sem0).wait()
    carry = process_chunk(0, buf0, carry)
    pltpu.async_copy(keys.at[pl.ds(row0(2), CHUNK)], buf0, sem0)
    pltpu.make_async_copy(keys.at[pl.ds(row0(1), CHUNK)], buf1, sem1).wait()
    carry = process_chunk(1, buf1, carry)
    pltpu.make_async_copy(keys.at[pl.ds(row0(2), CHUNK)], buf0, sem0).wait()
    carry = process_chunk(2, buf0, carry)

    best_t, best_i = carry
    for q in range(Q):
        res_t[q, :] = best_t[q]
        res_i[q, :] = best_i[q]
    pltpu.sync_copy(res_t, out_t.at[wid])
    pltpu.sync_copy(res_i, out_i.at[wid])


def _run_sc(keys, qsplat):
    mesh = plsc.VectorSubcoreMesh(core_axis_name="c", subcore_axis_name="s",
                                  num_cores=NC, num_subcores=NS)
    f = pl.kernel(
        _sc_body,
        out_type=(
            jax.ShapeDtypeStruct((NW, Q, L), jnp.float32),
            jax.ShapeDtypeStruct((NW, Q, L), jnp.int32),
        ),
        mesh=mesh,
        scratch_types=[
            pltpu.VMEM((CHUNK, D), jnp.float32),
            pltpu.VMEM((CHUNK, D), jnp.float32),
            pltpu.VMEM((Q * D * L,), jnp.float32),
            pltpu.VMEM((Q, L), jnp.float32),
            pltpu.VMEM((Q, L), jnp.int32),
            pltpu.SemaphoreType.DMA,
            pltpu.SemaphoreType.DMA,
        ],
        compiler_params=pltpu.CompilerParams(
            needs_layout_passes=False, use_tc_tiling_on_sc=False),
    )
    return f(keys, qsplat)


@jax.jit
def kernel(queries, keys):
    qn = queries / jnp.maximum(
        jnp.linalg.norm(queries, axis=-1, keepdims=True), EPS)
    qsplat = jnp.broadcast_to(qn.reshape(Q, D, 1), (Q, D, L)).reshape(-1)

    eye4 = jnp.eye(P4, dtype=jnp.float32)
    qn4 = jnp.einsum("qd,pr->qprd", qn, eye4).reshape(QP, W4)
    ones4 = jnp.einsum("pr,d->prd", eye4,
                       jnp.ones((D,), jnp.float32)).reshape(P4, W4)

    t_c, i_c = _run_sc(keys, qsplat)
    tc_v, tc_i = _run_tc(qn4, ones4, keys.reshape(NR, W4))

    sc_sims = jnp.sign(t_c) * jnp.sqrt(jnp.abs(t_c))
    sc_sims = sc_sims.transpose(1, 0, 2).reshape(Q, NW * L)
    sc_idx = i_c.transpose(1, 0, 2).reshape(Q, NW * L)
    sims = jnp.concatenate([sc_sims, tc_v.reshape(Q, P4)], axis=1)
    idx = jnp.concatenate([sc_idx, tc_i.reshape(Q, P4)], axis=1)
    vals = jnp.max(sims, axis=1)
    at_max = sims == vals[:, None]
    best_idx = jnp.min(jnp.where(at_max, idx, N), axis=1)
    return vals, best_idx.astype(jnp.int32)


# speedup vs baseline: 1.4861x; 1.0587x over previous
import functools

import jax
import jax.numpy as jnp
from jax import lax
from jax.experimental import pallas as pl
from jax.experimental.pallas import tpu as pltpu
from jax.experimental.pallas import tpu_sc as plsc

N = 1_000_000
D = 32
Q = 8
NC = 2
NS = 16
NW = NC * NS
L = 16

CHUNK = 1024
TPC = 3
N_SC = NW * TPC * CHUNK
N_TC = N - N_SC
TILE = 64
EPS = 1e-8
EPS2 = EPS * EPS
NEG_INF = float("-inf")


P4 = 4
W4 = P4 * D
NR = N // P4
BK4 = 8192
G_TC = (N_TC // P4 + BK4 - 1) // BK4
QP = Q * P4


def _tc_body(qn4_ref, ones4_ref, keys_ref, out_v, out_i, best_v, best_i):
    pi = pl.program_id(0)

    @pl.when(pi == 0)
    def _init():
        best_v[...] = jnp.full((QP, 1), NEG_INF, jnp.float32)
        best_i[...] = jnp.zeros((QP, 1), jnp.int32)

    kr = keys_ref[...]
    d4 = lax.dot_general(qn4_ref[...], kr, (((1,), (1,)), ((), ())),
                         preferred_element_type=jnp.float32)
    ksq = kr * kr
    s4 = lax.dot_general(ones4_ref[...], ksq, (((1,), (1,)), ((), ())),
                         preferred_element_type=jnp.float32)
    rs = lax.rsqrt(jnp.maximum(s4, EPS2))
    sims = d4 * jnp.tile(rs, (Q, 1))

    col = lax.broadcasted_iota(jnp.int32, (QP, BK4), 1)
    prow = lax.broadcasted_iota(jnp.int32, (QP, BK4), 0) % P4
    row = pi * (BK4 * P4) + col * P4 + prow
    sims = jnp.where(row < N_TC, sims, NEG_INF)

    m = jnp.max(sims, axis=1, keepdims=True)
    cand = jnp.where(sims == m, row, N)
    ci = jnp.min(cand, axis=1, keepdims=True)

    upd = m > best_v[...]
    best_v[...] = jnp.where(upd, m, best_v[...])
    best_i[...] = jnp.where(upd, ci, best_i[...])

    @pl.when(pi == G_TC - 1)
    def _out():
        out_v[...] = best_v[...]
        out_i[...] = best_i[...]


def _run_tc(qn4, ones4, keys_r):
    return pl.pallas_call(
        _tc_body,
        grid=(G_TC,),
        in_specs=[
            pl.BlockSpec((QP, W4), lambda i: (0, 0)),
            pl.BlockSpec((P4, W4), lambda i: (0, 0)),
            pl.BlockSpec((BK4, W4), lambda i: (i, 0)),
        ],
        out_specs=[
            pl.BlockSpec((QP, 1), lambda i: (0, 0)),
            pl.BlockSpec((QP, 1), lambda i: (0, 0)),
        ],
        out_shape=[
            jax.ShapeDtypeStruct((QP, 1), jnp.float32),
            jax.ShapeDtypeStruct((QP, 1), jnp.int32),
        ],
        scratch_shapes=[
            pltpu.VMEM((QP, 1), jnp.float32),
            pltpu.VMEM((QP, 1), jnp.int32),
        ],
    )(qn4, ones4, keys_r)


def _sc_body(keys, qsplat, out_t, out_i, buf0, buf1, qv, res_t, res_i,
             sem0, sem1):
    cid = lax.axis_index("c")
    sid = lax.axis_index("s")
    wid = cid * NS + sid

    pltpu.sync_copy(qsplat, qv)

    def row0(t):
        return N_TC + (wid + NW * t) * CHUNK

    iota = lax.iota(jnp.int32, L)

    def process_chunk(t, buf, carry):
        base_row = row0(t)

        def tile_body(tile, carry):
            best_t, best_i = carry
            rows = [tile * TILE + j * L + iota for j in range(TILE // L)]
            nj = len(rows)

            accs = [jnp.zeros((L,), jnp.float32) for _ in range(nj * (Q + 1))]
            for d in range(D):
                col = jnp.full((L,), d, jnp.int32)
                v = [plsc.load_gather(buf, [r, col]) for r in rows]
                for q in range(Q):
                    s = qv[pl.ds((q * D + d) * L, L)]
                    for j in range(nj):
                        accs[j * (Q + 1) + q] = accs[j * (Q + 1) + q] + v[j] * s
                for j in range(nj):
                    accs[j * (Q + 1) + Q] = accs[j * (Q + 1) + Q] + v[j] * v[j]

            best_t = list(best_t)
            best_i = list(best_i)
            for j in range(nj):
                rcp = 1.0 / jnp.maximum(accs[j * (Q + 1) + Q], EPS2)
                idx_vec = base_row + rows[j]
                for q in range(Q):
                    dot = accs[j * (Q + 1) + q]
                    tval = dot * jnp.abs(dot) * rcp
                    better = tval > best_t[q]
                    best_t[q] = jnp.where(better, tval, best_t[q])
                    best_i[q] = jnp.where(better, idx_vec, best_i[q])
            return (tuple(best_t), tuple(best_i))

        return lax.fori_loop(0, CHUNK // TILE, tile_body, carry)

    best_t = tuple(jnp.full((L,), NEG_INF, jnp.float32) for _ in range(Q))
    best_i = tuple(jnp.zeros((L,), jnp.int32) for _ in range(Q))
    carry = (best_t, best_i)

    pltpu.async_copy(keys.at[pl.ds(row0(0), CHUNK)], buf0, sem0)
    pltpu.async_copy(keys.at[pl.ds(row0(1), CHUNK)], buf1, sem1)
    pltpu.make_async_copy(keys.at[pl.ds(row0(0), CHUNK)], buf0, sem0).wait()
    carry = process_chunk(0, buf0, carry)
    pltpu.async_copy(keys.at[pl.ds(row0(2), CHUNK)], buf0, sem0)
    pltpu.make_async_copy(keys.at[pl.ds(row0(1), CHUNK)], buf1, sem1).wait()
    carry = process_chunk(1, buf1, carry)
    pltpu.make_async_copy(keys.at[pl.ds(row0(2), CHUNK)], buf0, sem0).wait()
    carry = process_chunk(2, buf0, carry)

    best_t, best_i = carry
    for q in range(Q):
        res_t[q, :] = best_t[q]
        res_i[q, :] = best_i[q]
    pltpu.sync_copy(res_t, out_t.at[wid])
    pltpu.sync_copy(res_i, out_i.at[wid])


def _run_sc(keys, qsplat):
    mesh = plsc.VectorSubcoreMesh(core_axis_name="c", subcore_axis_name="s",
                                  num_cores=NC, num_subcores=NS)
    f = pl.kernel(
        _sc_body,
        out_type=(
            jax.ShapeDtypeStruct((NW, Q, L), jnp.float32),
            jax.ShapeDtypeStruct((NW, Q, L), jnp.int32),
        ),
        mesh=mesh,
        scratch_types=[
            pltpu.VMEM((CHUNK, D), jnp.float32),
            pltpu.VMEM((CHUNK, D), jnp.float32),
            pltpu.VMEM((Q * D * L,), jnp.float32),
            pltpu.VMEM((Q, L), jnp.float32),
            pltpu.VMEM((Q, L), jnp.int32),
            pltpu.SemaphoreType.DMA,
            pltpu.SemaphoreType.DMA,
        ],
        compiler_params=pltpu.CompilerParams(
            needs_layout_passes=False, use_tc_tiling_on_sc=False),
    )
    return f(keys, qsplat)


@jax.jit
def kernel(queries, keys):
    qn = queries / jnp.maximum(
        jnp.linalg.norm(queries, axis=-1, keepdims=True), EPS)
    qsplat = jnp.broadcast_to(qn.reshape(Q, D, 1), (Q, D, L)).reshape(-1)

    eye4 = jnp.eye(P4, dtype=jnp.float32)
    qn4 = jnp.einsum("qd,pr->qprd", qn, eye4).reshape(QP, W4)
    ones4 = jnp.einsum("pr,d->prd", eye4,
                       jnp.ones((D,), jnp.float32)).reshape(P4, W4)

    t_c, i_c = _run_sc(keys, qsplat)
    tc_v, tc_i = _run_tc(qn4, ones4, keys.reshape(NR, W4))

    sc_sims = jnp.sign(t_c) * jnp.sqrt(jnp.abs(t_c))
    sc_sims = sc_sims.transpose(1, 0, 2).reshape(Q, NW * L)
    sc_idx = i_c.transpose(1, 0, 2).reshape(Q, NW * L)
    sims = jnp.concatenate([sc_sims, tc_v.reshape(Q, P4)], axis=1)
    idx = jnp.concatenate([sc_idx, tc_i.reshape(Q, P4)], axis=1)
    vals = jnp.max(sims, axis=1)
    at_max = sims == vals[:, None]
    best_idx = jnp.min(jnp.where(at_max, idx, N), axis=1)
    return vals, best_idx.astype(jnp.int32)
